# baseline (speedup 1.0000x reference)
import functools

import jax
import jax.numpy as jnp
from jax import lax
from jax.experimental import pallas as pl
from jax.experimental.pallas import tpu as pltpu
from jax.experimental.pallas import tpu_sc as plsc

_D = 64
_B = 16384
_NEG = 20
_SLOTS = _NEG + 2
_ROWS = _B * _SLOTS
_NC = 2
_NS = 16
_NW = _NC * _NS
_RPW = _ROWS // _NW
_CH = 128
_NCH = _RPW // _CH

_BS = 2048


def _gather_rows(idx2d, table):
    mesh = plsc.VectorSubcoreMesh(core_axis_name="c", subcore_axis_name="s")

    @functools.partial(
        pl.kernel,
        mesh=mesh,
        out_type=jax.ShapeDtypeStruct((_ROWS, _D), jnp.float32),
        scratch_types=[
            pltpu.VMEM((_NCH, _CH), jnp.int32),
            pltpu.VMEM((_CH, _D), jnp.float32),
            pltpu.SemaphoreType.DMA,
        ],
        compiler_params=pltpu.CompilerParams(use_tc_tiling_on_sc=False),
    )
    def k(idx_hbm, table_hbm, out_hbm, idx_v, rows_v, sem):
        wid = lax.axis_index("s") * _NC + lax.axis_index("c")
        pltpu.sync_copy(idx_hbm.at[pl.ds(wid * _NCH, _NCH)], idx_v)
        row0 = wid * _RPW

        def body(j, carry):
            pltpu.async_copy(table_hbm.at[idx_v.at[j]], rows_v, sem).wait()
            pltpu.sync_copy(rows_v, out_hbm.at[pl.ds(row0 + j * _CH, _CH)])
            return carry

        lax.fori_loop(0, _NCH, body, 0)

    return k(idx2d, table)


def _score_body(g_ref, d_ref, o_ref):
    i = pl.program_id(0)
    g = g_ref[...]
    gu = g[:, 0, :]
    gv = g[:, 1, :]
    gneg = g[:, 2:, :]
    ud = jnp.dot(gu, d_ref[...], preferred_element_type=jnp.float32)
    pos = jnp.clip(jnp.sum(ud * gv, axis=1), -10.0, 10.0)
    neg = jnp.clip(jnp.sum(gneg * ud[:, None, :], axis=2), -10.0, 10.0)
    t = (jnp.sum(jax.nn.softplus(-pos)) + jnp.sum(jax.nn.softplus(neg))) * (
        1.0 / _B)

    t2 = t[None, None]

    @pl.when(i == 0)
    def _():
        o_ref[...] = t2

    @pl.when(i > 0)
    def _():
        o_ref[...] += t2


def _score(g3, diag):
    out = pl.pallas_call(
        _score_body,
        grid=(_B // _BS,),
        in_specs=[
            pl.BlockSpec((_BS, _SLOTS, _D), lambda i: (i, 0, 0)),
            pl.BlockSpec((_D, _D), lambda i: (0, 0)),
        ],
        out_specs=pl.BlockSpec((1, 1), lambda i: (0, 0)),
        out_shape=jax.ShapeDtypeStruct((1, 1), jnp.float32),
    )(g3, diag)
    return out[0, 0]


def kernel(pos_u, pos_v, neg_v, diag, u_weight):
    idx = jnp.concatenate(
        [pos_u[:, None], pos_v[:, None], neg_v], axis=1)
    idx = idx.astype(jnp.int32).reshape(_NW * _NCH, _CH)
    g = _gather_rows(idx, u_weight)
    return _score(g.reshape(_B, _SLOTS, _D), diag)
